# baseline (speedup 1.0000x reference)
import functools

import jax
import jax.numpy as jnp
from jax import lax
from jax.experimental import pallas as pl
from jax.experimental.pallas import tpu as pltpu
from jax.experimental.pallas import tpu_sc as plsc

HID = 128
GAMMA = 12.0
ERANGE = (12.0 + 2.0) / HID
PI = 3.141592653589793
PHASE_SCALE = PI / ERANGE

NC = 2
NS = 16
NW = NC * NS
LANES = 16
NBLK = 4


_COS_C = (9.9999999988e-01, -4.9999999850e-01, 4.1666663478e-02,
          -1.3888862974e-03, 2.4800551329e-05, -2.7534767425e-07,
          2.0603329868e-09, -9.7217335580e-12)
_SIN_C = (9.9999999999e-01, -1.6666666658e-01, 8.3333331432e-03,
          -1.9841254393e-04, 2.7556701964e-06, -2.5038681784e-08,
          1.5896473967e-10, -6.6101222328e-13)


def _trig_body(rel_ref, trig_ref):
    phase = rel_ref[...] * jnp.float32(PHASE_SCALE)
    t = phase * phase
    c = jnp.float32(_COS_C[-1])
    s = jnp.float32(_SIN_C[-1])
    for cc, sc in zip(_COS_C[-2::-1], _SIN_C[-2::-1]):
        c = c * t + jnp.float32(cc)
        s = s * t + jnp.float32(sc)
    trig_ref[:, :HID] = c
    trig_ref[:, HID:] = phase * s


def _trig_table(rel_emb):
    n, h = rel_emb.shape
    return pl.pallas_call(
        _trig_body,
        out_shape=jax.ShapeDtypeStruct((n, 2 * h), jnp.float32),
    )(rel_emb)


def _score_block(ht_v, c_v, ts_v, out_v, blk, bs):
    lane16 = lax.broadcasted_iota(jnp.int32, (LANES,), 0) * jnp.int32(LANES)

    def group_step(g, carry):
        def sample_step(k, carry2):
            i = g * LANES + k
            chunks = []
            for j in range(HID // LANES):
                sl = pl.ds(j * LANES, LANES)
                sl2 = pl.ds(HID + j * LANES, LANES)
                rh = ht_v[i, sl]
                ih = ht_v[i, sl2]
                rt = ht_v[bs + i, sl]
                it = ht_v[bs + i, sl2]
                c = c_v[i, sl]
                s = c_v[i, sl2]
                re = rh * c - ih * s - rt
                im = rh * s + ih * c - it
                x = re * re + (im * im + jnp.float32(1e-30))
                yi = jnp.int32(0x5F1FFFF9) - lax.shift_right_logical(
                    plsc.bitcast(x, jnp.int32), jnp.int32(1))
                y = plsc.bitcast(yi, jnp.float32)
                t = x * y
                chunks.append(t * (jnp.float32(2.38924456) - t * y))
            while len(chunks) > 1:
                chunks = [a + b for a, b in zip(chunks[::2], chunks[1::2])]
            plsc.store_scatter(ts_v, [lane16 + k], chunks[0])
            return carry2

        lax.fori_loop(0, LANES, sample_step, 0)
        rows = [ts_v[pl.ds(r * LANES, LANES)] for r in range(LANES)]
        while len(rows) > 1:
            rows = [a + b for a, b in zip(rows[::2], rows[1::2])]
        out_v[pl.ds(blk * bs + g * LANES, LANES)] = (
            jnp.float32(GAMMA) - jnp.float32(0.703952253) * rows[0])
        return carry

    lax.fori_loop(0, bs // LANES, group_step, 0)


def _sc_body(bpw, ent_hbm, trig_hbm, idxht_hbm, idxr_hbm,
             out_hbm, idxht_v, idxr_v,
             ht0_v, ht1_v, c0_v, c1_v, ts_v, out_v,
             sem0, sem1, semi):
    bs = bpw // NBLK
    wid = lax.axis_index("s") * NC + lax.axis_index("c")
    base = wid * bpw
    ch = pltpu.async_copy(idxht_hbm.at[pl.ds(base * 2, bpw * 2)], idxht_v, semi)
    cr = pltpu.async_copy(idxr_hbm.at[pl.ds(base, bpw)], idxr_v, semi)
    ch.wait()
    cr.wait()

    htb = [ht0_v, ht1_v]
    cb = [c0_v, c1_v]
    sems = [sem0, sem1]

    def fire(blk):
        b = blk % 2
        return (
            pltpu.async_copy(
                ent_hbm.at[idxht_v.at[pl.ds(blk * 2 * bs, 2 * bs)]],
                htb[b], sems[b]),
            pltpu.async_copy(
                trig_hbm.at[idxr_v.at[pl.ds(blk * bs, bs)]],
                cb[b], sems[b]),
        )

    inflight = fire(0)
    for blk in range(NBLK):
        cur = inflight
        if blk + 1 < NBLK:
            inflight = fire(blk + 1)
        for c in cur:
            c.wait()
        b = blk % 2
        _score_block(htb[b], cb[b], ts_v, out_v, blk, bs)

    pltpu.sync_copy(out_v, out_hbm.at[pl.ds(base, bpw)])


def _sc_score(ent_emb, trig_t, idx_ht, idx_r):
    batch = idx_r.shape[0]
    assert batch % (8 * NW) == 0
    bpw = batch // NW
    bs = bpw // NBLK
    dent = ent_emb.shape[1]
    mesh = plsc.VectorSubcoreMesh(core_axis_name="c", subcore_axis_name="s")
    kfn = functools.partial(
        pl.kernel,
        mesh=mesh,
        compiler_params=pltpu.CompilerParams(needs_layout_passes=False),
        out_type=jax.ShapeDtypeStruct((batch,), jnp.float32),
        scratch_types=[
            pltpu.VMEM((2 * bpw,), jnp.int32),
            pltpu.VMEM((bpw,), jnp.int32),
            pltpu.VMEM((2 * bs, dent), jnp.float32),
            pltpu.VMEM((2 * bs, dent), jnp.float32),
            pltpu.VMEM((bs, 2 * HID), jnp.float32),
            pltpu.VMEM((bs, 2 * HID), jnp.float32),
            pltpu.VMEM((LANES * LANES,), jnp.float32),
            pltpu.VMEM((bpw,), jnp.float32),
            pltpu.SemaphoreType.DMA,
            pltpu.SemaphoreType.DMA,
            pltpu.SemaphoreType.DMA,
        ],
    )(functools.partial(_sc_body, bpw))
    return kfn(ent_emb, trig_t, idx_ht, idx_r)


def kernel(sample, ent_emb, rel_emb):
    trig_t = _trig_table(rel_emb)
    idx = sample.astype(jnp.int32)
    batch = idx.shape[0]
    bpw = batch // NW
    bs = bpw // NBLK
    ih = idx[:, 0].reshape(NW, NBLK, bs)
    it = idx[:, 2].reshape(NW, NBLK, bs)
    idx_ht = jnp.concatenate([ih, it], axis=2).reshape(-1)
    score = _sc_score(ent_emb, trig_t, idx_ht, idx[:, 1])
    return score[:, None]


# speedup vs baseline: 1.0326x; 1.0326x over previous
import functools

import jax
import jax.numpy as jnp
from jax import lax
from jax.experimental import pallas as pl
from jax.experimental.pallas import tpu as pltpu
from jax.experimental.pallas import tpu_sc as plsc

HID = 128
GAMMA = 12.0
ERANGE = (12.0 + 2.0) / HID
PI = 3.141592653589793
PHASE_SCALE = PI / ERANGE

NC = 2
NS = 16
NW = NC * NS
LANES = 16
NBLK = 4


_COS_C = (9.9999999988e-01, -4.9999999850e-01, 4.1666663478e-02,
          -1.3888862974e-03, 2.4800551329e-05, -2.7534767425e-07,
          2.0603329868e-09, -9.7217335580e-12)
_SIN_C = (9.9999999999e-01, -1.6666666658e-01, 8.3333331432e-03,
          -1.9841254393e-04, 2.7556701964e-06, -2.5038681784e-08,
          1.5896473967e-10, -6.6101222328e-13)


def _trig_body(rel_ref, trig_ref):
    phase = rel_ref[...] * jnp.float32(PHASE_SCALE)
    t = phase * phase
    c = jnp.float32(_COS_C[-1])
    s = jnp.float32(_SIN_C[-1])
    for cc, sc in zip(_COS_C[-2::-1], _SIN_C[-2::-1]):
        c = c * t + jnp.float32(cc)
        s = s * t + jnp.float32(sc)
    trig_ref[:, :HID] = c
    trig_ref[:, HID:] = phase * s


def _trig_table(rel_emb):
    n, h = rel_emb.shape
    return pl.pallas_call(
        _trig_body,
        out_shape=jax.ShapeDtypeStruct((n, 2 * h), jnp.float32),
    )(rel_emb)


def _score_block(h_v, t_v, c_v, ts_v, out_v, blk, bs):
    lane16 = lax.broadcasted_iota(jnp.int32, (LANES,), 0) * jnp.int32(LANES)

    def group_step(g, carry):
        def sample_step(k, carry2):
            i = g * LANES + k
            chunks = []
            for j in range(HID // LANES):
                sl = pl.ds(j * LANES, LANES)
                sl2 = pl.ds(HID + j * LANES, LANES)
                rh = h_v[i, sl]
                ih = h_v[i, sl2]
                rt = t_v[i, sl]
                it = t_v[i, sl2]
                c = c_v[i, sl]
                s = c_v[i, sl2]
                re = rh * c - ih * s - rt
                im = rh * s + ih * c - it
                x = re * re + (im * im + jnp.float32(1e-30))
                yi = jnp.int32(0x5F1FFFF9) - lax.shift_right_logical(
                    plsc.bitcast(x, jnp.int32), jnp.int32(1))
                y = plsc.bitcast(yi, jnp.float32)
                t = x * y
                chunks.append(t * (jnp.float32(2.38924456) - t * y))
            while len(chunks) > 1:
                chunks = [a + b for a, b in zip(chunks[::2], chunks[1::2])]
            plsc.store_scatter(ts_v, [lane16 + k], chunks[0])
            return carry2

        lax.fori_loop(0, LANES, sample_step, 0)
        rows = [ts_v[pl.ds(r * LANES, LANES)] for r in range(LANES)]
        while len(rows) > 1:
            rows = [a + b for a, b in zip(rows[::2], rows[1::2])]
        out_v[pl.ds(blk * bs + g * LANES, LANES)] = (
            jnp.float32(GAMMA) - jnp.float32(0.703952253) * rows[0])
        return carry

    lax.fori_loop(0, bs // LANES, group_step, 0)


def _sc_body(bpw, ent_hbm, trig_hbm, idxh_hbm, idxr_hbm, idxt_hbm,
             out_hbm, idxh_v, idxr_v, idxt_v,
             h0_v, h1_v, t0_v, t1_v, c0_v, c1_v, ts_v, out_v,
             sem0, sem1, semi):
    bs = bpw // NBLK
    wid = lax.axis_index("s") * NC + lax.axis_index("c")
    base = wid * bpw
    ci = pltpu.async_copy(idxh_hbm.at[pl.ds(base, bpw)], idxh_v, semi)
    cr = pltpu.async_copy(idxr_hbm.at[pl.ds(base, bpw)], idxr_v, semi)
    ct = pltpu.async_copy(idxt_hbm.at[pl.ds(base, bpw)], idxt_v, semi)
    ci.wait()
    cr.wait()
    ct.wait()

    hb = [h0_v, h1_v]
    tb = [t0_v, t1_v]
    cb = [c0_v, c1_v]
    sems = [sem0, sem1]

    def fire(blk):
        b = blk % 2
        s = pl.ds(blk * bs, bs)
        return (
            pltpu.async_copy(ent_hbm.at[idxh_v.at[s]], hb[b], sems[b]),
            pltpu.async_copy(ent_hbm.at[idxt_v.at[s]], tb[b], sems[b]),
            pltpu.async_copy(trig_hbm.at[idxr_v.at[s]], cb[b], sems[b]),
        )

    inflight = fire(0)
    for blk in range(NBLK):
        cur = inflight
        if blk + 1 < NBLK:
            inflight = fire(blk + 1)
        for c in cur:
            c.wait()
        b = blk % 2
        _score_block(hb[b], tb[b], cb[b], ts_v, out_v, blk, bs)

    pltpu.sync_copy(out_v, out_hbm.at[pl.ds(base, bpw)])


def _sc_score(ent_emb, trig_t, idx_h, idx_r, idx_t):
    batch = idx_h.shape[0]
    assert batch % (8 * NW) == 0
    bpw = batch // NW
    bs = bpw // NBLK
    dent = ent_emb.shape[1]
    mesh = plsc.VectorSubcoreMesh(core_axis_name="c", subcore_axis_name="s")
    kfn = functools.partial(
        pl.kernel,
        mesh=mesh,
        compiler_params=pltpu.CompilerParams(needs_layout_passes=False),
        out_type=jax.ShapeDtypeStruct((batch,), jnp.float32),
        scratch_types=[
            pltpu.VMEM((bpw,), jnp.int32),
            pltpu.VMEM((bpw,), jnp.int32),
            pltpu.VMEM((bpw,), jnp.int32),
            pltpu.VMEM((bs, dent), jnp.float32),
            pltpu.VMEM((bs, dent), jnp.float32),
            pltpu.VMEM((bs, dent), jnp.float32),
            pltpu.VMEM((bs, dent), jnp.float32),
            pltpu.VMEM((bs, 2 * HID), jnp.float32),
            pltpu.VMEM((bs, 2 * HID), jnp.float32),
            pltpu.VMEM((LANES * LANES,), jnp.float32),
            pltpu.VMEM((bpw,), jnp.float32),
            pltpu.SemaphoreType.DMA,
            pltpu.SemaphoreType.DMA,
            pltpu.SemaphoreType.DMA,
        ],
    )(functools.partial(_sc_body, bpw))
    return kfn(ent_emb, trig_t, idx_h, idx_r, idx_t)


def kernel(sample, ent_emb, rel_emb):
    trig_t = _trig_table(rel_emb)
    idx = sample.astype(jnp.int32)
    score = _sc_score(ent_emb, trig_t, idx[:, 0], idx[:, 1], idx[:, 2])
    return score[:, None]
